# baseline (speedup 1.0000x reference)
import functools

import jax
import jax.numpy as jnp
from jax.experimental import pallas as pl
from jax.experimental.pallas import tpu as pltpu

_T = 10.0
_NEG_BIG = 1e12
_F32 = jnp.float32
_BF16 = jnp.bfloat16
_HI = jax.lax.Precision.HIGHEST


def _bdot(a, b, dims):
    return jax.lax.dot_general(a.astype(_BF16), b.astype(_BF16), (dims, ((), ())),
                               preferred_element_type=_F32)


def _scores(x, sf_ref, base, kb, k_total):
    valid_c = (jax.lax.broadcasted_iota(jnp.int32, (kb, 1), 0) + base) < k_total
    sf = jnp.where(valid_c, sf_ref[...], 0.0)
    pen = jnp.where(
        (jax.lax.broadcasted_iota(jnp.int32, (1, kb), 1) + base) < k_total,
        0.0, _NEG_BIG)
    xn = jnp.sum(x * x, axis=1, keepdims=True)
    sfn = jnp.transpose(jnp.sum(sf * sf, axis=1, keepdims=True))
    dot2 = _bdot(x + x, sf, ((1,), (1,)))
    sq = (xn + (sfn + pen)) - dot2
    return -_T * jnp.sqrt(jnp.maximum(sq, 1e-12)), valid_c


def _pass1_body(x_ref, sf_ref, m_ref, l_ref, m_s, l_s,
                *, kb, k_total, n_blocks):
    j = pl.program_id(0)

    @pl.when(j == 0)
    def _init():
        m_s[...] = jnp.full_like(m_s, -1e30)
        l_s[...] = jnp.zeros_like(l_s)

    s, _ = _scores(x_ref[...], sf_ref, j * kb, kb, k_total)
    m_old = m_s[...]
    m_new = jnp.maximum(m_old, jnp.max(s, axis=1, keepdims=True))
    alpha = jnp.exp(m_old - m_new)
    p = jnp.exp(s - m_new)
    m_s[...] = m_new
    l_s[...] = l_s[...] * alpha + jnp.sum(p, axis=1, keepdims=True)

    @pl.when(j == n_blocks - 1)
    def _finalize():
        m_ref[...] = m_s[...]
        l_ref[...] = l_s[...]


def _pass2_body(x_ref, sf_ref, df_ref, m_ref, l_ref, xt_ref, acc_s,
                *, kb, k_total, n_blocks):
    j = pl.program_id(0)

    @pl.when(j == 0)
    def _init():
        acc_s[...] = jnp.zeros_like(acc_s)

    s, valid_c = _scores(x_ref[...], sf_ref, j * kb, kb, k_total)
    w1 = jnp.exp(s - m_ref[...]) / l_ref[...]
    df = jnp.where(valid_c, df_ref[...], 0.0)
    acc_s[...] = acc_s[...] + _bdot(w1, df, ((1,), (0,)))

    @pl.when(j == n_blocks - 1)
    def _finalize():
        xt_ref[...] = acc_s[...]


def _stage2_body(xt_ref, df_ref, lab_ref, idx_ref, er_ref,
                 y_ref,
                 m_s, l_s, acc_s,
                 *, kb, k_total, n_blocks, n_labels):
    j = pl.program_id(0)
    base = j * kb

    @pl.when(j == 0)
    def _init():
        m_s[...] = jnp.full_like(m_s, -1e30)
        l_s[...] = jnp.zeros_like(l_s)
        acc_s[...] = jnp.zeros_like(acc_s)

    s, valid_c = _scores(xt_ref[...], df_ref, base, kb, k_total)
    lab = jnp.where(valid_c, lab_ref[...], 0.0)

    idx = idx_ref[0]
    lab_lanes = jax.lax.broadcasted_iota(jnp.int32, (2 * n_labels, kb), 0)
    oht2 = ((lab_lanes == idx) |
            (lab_lanes == idx + n_labels)).astype(_F32)
    factor = _bdot(er_ref[...], oht2, ((1,), (0,)))

    m_old = m_s[...]
    m_new = jnp.maximum(m_old, jnp.max(s, axis=1, keepdims=True))
    alpha = jnp.exp(m_old - m_new)
    p = jnp.exp(s - m_new) * factor
    m_s[...] = m_new
    l_s[...] = l_s[...] * alpha + jnp.sum(p, axis=1, keepdims=True)
    acc_s[...] = acc_s[...] * alpha + _bdot(p, lab, ((1,), (0,)))

    @pl.when(j == n_blocks - 1)
    def _finalize():
        y_ref[...] = acc_s[...] / l_s[...]


@jax.jit
def kernel(x, star_features, dataset_features, W, b, label_distances,
           star_sample_labels, dataset_label_idx):
    q, d = x.shape
    k_total = star_features.shape[0]
    n_labels = W.shape[1]
    kb = 2048 if k_total >= 2048 else 256
    n_blocks = pl.cdiv(k_total, kb)
    k_pad = n_blocks * kb

    idx3 = jnp.pad(dataset_label_idx, (0, k_pad - k_total),
                   constant_values=n_labels).reshape(n_blocks, 1, kb)

    m1, l1 = pl.pallas_call(
        functools.partial(_pass1_body, kb=kb, k_total=k_total,
                          n_blocks=n_blocks),
        grid=(n_blocks,),
        in_specs=[
            pl.BlockSpec((q, d), lambda j: (0, 0)),
            pl.BlockSpec((kb, d), lambda j: (j, 0)),
        ],
        out_specs=[
            pl.BlockSpec((q, 1), lambda j: (0, 0)),
            pl.BlockSpec((q, 1), lambda j: (0, 0)),
        ],
        out_shape=[
            jax.ShapeDtypeStruct((q, 1), _F32),
            jax.ShapeDtypeStruct((q, 1), _F32),
        ],
        scratch_shapes=[
            pltpu.VMEM((q, 1), _F32),
            pltpu.VMEM((q, 1), _F32),
        ],
    )(x, star_features)

    x_t = pl.pallas_call(
        functools.partial(_pass2_body, kb=kb, k_total=k_total,
                          n_blocks=n_blocks),
        grid=(n_blocks,),
        in_specs=[
            pl.BlockSpec((q, d), lambda j: (0, 0)),
            pl.BlockSpec((kb, d), lambda j: (j, 0)),
            pl.BlockSpec((kb, d), lambda j: (j, 0)),
            pl.BlockSpec((q, 1), lambda j: (0, 0)),
            pl.BlockSpec((q, 1), lambda j: (0, 0)),
        ],
        out_specs=pl.BlockSpec((q, d), lambda j: (0, 0)),
        out_shape=jax.ShapeDtypeStruct((q, d), _F32),
        scratch_shapes=[
            pltpu.VMEM((q, d), _F32),
        ],
    )(x, star_features, dataset_features, m1, l1)

    preds = x_t @ W + b
    closest = jnp.argmax(preds, axis=1)
    e_rows = jnp.exp(-_T * label_distances)[closest]
    er_hi = e_rows.astype(_BF16).astype(_F32)
    er2 = jnp.concatenate([er_hi, e_rows - er_hi], axis=1)

    y = pl.pallas_call(
        functools.partial(_stage2_body, kb=kb, k_total=k_total,
                          n_blocks=n_blocks, n_labels=n_labels),
        grid=(n_blocks,),
        in_specs=[
            pl.BlockSpec((q, d), lambda j: (0, 0)),
            pl.BlockSpec((kb, d), lambda j: (j, 0)),
            pl.BlockSpec((kb, n_labels), lambda j: (j, 0)),
            pl.BlockSpec((1, 1, kb), lambda j: (j, 0, 0)),
            pl.BlockSpec((q, 2 * n_labels), lambda j: (0, 0)),
        ],
        out_specs=pl.BlockSpec((q, n_labels), lambda j: (0, 0)),
        out_shape=jax.ShapeDtypeStruct((q, n_labels), _F32),
        scratch_shapes=[
            pltpu.VMEM((q, 1), _F32),
            pltpu.VMEM((q, 1), _F32),
            pltpu.VMEM((q, n_labels), _F32),
        ],
    )(x_t, dataset_features, star_sample_labels, idx3, er2)

    return y


# speedup vs baseline: 1.0843x; 1.0843x over previous
import functools

import jax
import jax.numpy as jnp
from jax.experimental import pallas as pl
from jax.experimental.pallas import tpu as pltpu

_T = 10.0
_NEG_BIG = 1e12
_F32 = jnp.float32
_BF16 = jnp.bfloat16
_HI = jax.lax.Precision.HIGHEST


def _bdot(a, b, dims):
    return jax.lax.dot_general(a.astype(_BF16), b.astype(_BF16), (dims, ((), ())),
                               preferred_element_type=_F32)


def _scores(x, sf_ref, base, kb, k_total):
    valid_c = (jax.lax.broadcasted_iota(jnp.int32, (kb, 1), 0) + base) < k_total
    sf = jnp.where(valid_c, sf_ref[...], 0.0)
    pen = jnp.where(
        (jax.lax.broadcasted_iota(jnp.int32, (1, kb), 1) + base) < k_total,
        0.0, _NEG_BIG)
    xn = jnp.sum(x * x, axis=1, keepdims=True)
    sfn = jnp.transpose(jnp.sum(sf * sf, axis=1, keepdims=True))
    dot2 = _bdot(x + x, sf, ((1,), (1,)))
    sq = (xn + (sfn + pen)) - dot2
    return -_T * jnp.sqrt(jnp.maximum(sq, 1e-12)), valid_c


def _stage1_body(x_ref, sf_ref, df_ref, xt_ref, m_s, l_s, acc_s,
                 *, kb, k_total, n_blocks):
    j = pl.program_id(0)
    jj = jax.lax.rem(j, n_blocks)

    @pl.when(j == 0)
    def _init():
        m_s[...] = jnp.full_like(m_s, -1e30)
        l_s[...] = jnp.zeros_like(l_s)
        acc_s[...] = jnp.zeros_like(acc_s)

    s, valid_c = _scores(x_ref[...], sf_ref, jj * kb, kb, k_total)

    @pl.when(j < n_blocks)
    def _pass1():
        m_old = m_s[...]
        m_new = jnp.maximum(m_old, jnp.max(s, axis=1, keepdims=True))
        alpha = jnp.exp(m_old - m_new)
        p = jnp.exp(s - m_new)
        m_s[...] = m_new
        l_s[...] = l_s[...] * alpha + jnp.sum(p, axis=1, keepdims=True)

    @pl.when(j >= n_blocks)
    def _pass2():
        w1 = jnp.exp(s - m_s[...]) / l_s[...]
        df = jnp.where(valid_c, df_ref[...], 0.0)
        acc_s[...] = acc_s[...] + _bdot(w1, df, ((1,), (0,)))

    @pl.when(j == 2 * n_blocks - 1)
    def _finalize():
        xt_ref[...] = acc_s[...]


def _stage2_body(xt_ref, df_ref, lab_ref, idx_ref, er_ref,
                 y_ref,
                 m_s, l_s, acc_s,
                 *, kb, k_total, n_blocks, n_labels):
    j = pl.program_id(0)
    base = j * kb

    @pl.when(j == 0)
    def _init():
        m_s[...] = jnp.full_like(m_s, -1e30)
        l_s[...] = jnp.zeros_like(l_s)
        acc_s[...] = jnp.zeros_like(acc_s)

    s, valid_c = _scores(xt_ref[...], df_ref, base, kb, k_total)
    lab = jnp.where(valid_c, lab_ref[...], 0.0)

    idx = idx_ref[0]
    lab_lanes = jax.lax.broadcasted_iota(jnp.int32, (2 * n_labels, kb), 0)
    oht2 = ((lab_lanes == idx) |
            (lab_lanes == idx + n_labels)).astype(_F32)
    factor = _bdot(er_ref[...], oht2, ((1,), (0,)))

    m_old = m_s[...]
    m_new = jnp.maximum(m_old, jnp.max(s, axis=1, keepdims=True))
    alpha = jnp.exp(m_old - m_new)
    p = jnp.exp(s - m_new) * factor
    m_s[...] = m_new
    l_s[...] = l_s[...] * alpha + jnp.sum(p, axis=1, keepdims=True)
    acc_s[...] = acc_s[...] * alpha + _bdot(p, lab, ((1,), (0,)))

    @pl.when(j == n_blocks - 1)
    def _finalize():
        y_ref[...] = acc_s[...] / l_s[...]


@jax.jit
def kernel(x, star_features, dataset_features, W, b, label_distances,
           star_sample_labels, dataset_label_idx):
    q, d = x.shape
    k_total = star_features.shape[0]
    n_labels = W.shape[1]
    kb = 2048 if k_total >= 2048 else 256
    n_blocks = pl.cdiv(k_total, kb)
    k_pad = n_blocks * kb

    idx3 = jnp.pad(dataset_label_idx, (0, k_pad - k_total),
                   constant_values=n_labels).reshape(n_blocks, 1, kb)

    x_t = pl.pallas_call(
        functools.partial(_stage1_body, kb=kb, k_total=k_total,
                          n_blocks=n_blocks),
        grid=(2 * n_blocks,),
        in_specs=[
            pl.BlockSpec((q, d), lambda j: (0, 0)),
            pl.BlockSpec((kb, d), lambda j: (jax.lax.rem(j, n_blocks), 0)),
            pl.BlockSpec((kb, d), lambda j: (jax.lax.rem(j, n_blocks), 0)),
        ],
        out_specs=pl.BlockSpec((q, d), lambda j: (0, 0)),
        out_shape=jax.ShapeDtypeStruct((q, d), _F32),
        scratch_shapes=[
            pltpu.VMEM((q, 1), _F32),
            pltpu.VMEM((q, 1), _F32),
            pltpu.VMEM((q, d), _F32),
        ],
    )(x, star_features, dataset_features)

    preds = x_t @ W + b
    closest = jnp.argmax(preds, axis=1)
    e_rows = jnp.exp(-_T * label_distances)[closest]
    er_hi = e_rows.astype(_BF16).astype(_F32)
    er2 = jnp.concatenate([er_hi, e_rows - er_hi], axis=1)

    y = pl.pallas_call(
        functools.partial(_stage2_body, kb=kb, k_total=k_total,
                          n_blocks=n_blocks, n_labels=n_labels),
        grid=(n_blocks,),
        in_specs=[
            pl.BlockSpec((q, d), lambda j: (0, 0)),
            pl.BlockSpec((kb, d), lambda j: (j, 0)),
            pl.BlockSpec((kb, n_labels), lambda j: (j, 0)),
            pl.BlockSpec((1, 1, kb), lambda j: (j, 0, 0)),
            pl.BlockSpec((q, 2 * n_labels), lambda j: (0, 0)),
        ],
        out_specs=pl.BlockSpec((q, n_labels), lambda j: (0, 0)),
        out_shape=jax.ShapeDtypeStruct((q, n_labels), _F32),
        scratch_shapes=[
            pltpu.VMEM((q, 1), _F32),
            pltpu.VMEM((q, 1), _F32),
            pltpu.VMEM((q, n_labels), _F32),
        ],
    )(x_t, dataset_features, star_sample_labels, idx3, er2)

    return y
